# baseline (speedup 1.0000x reference)
import jax
import jax.numpy as jnp
from jax import lax
from jax.experimental import pallas as pl
from jax.experimental.pallas import tpu as pltpu
from jax.experimental.pallas import tpu_sc as plsc

_ENT_ROWS = 1000000
_SPLIT = 500000

_NC, _NS = 2, 16
_NW = _NC * _NS
_SC_PIECE = 504
_SC_NPIECE = 31
_SC_CHUNK = _SC_PIECE * _SC_NPIECE
_SC_TAIL = (_ENT_ROWS - _SPLIT) - _NW * _SC_CHUNK

_REL_SPLIT = (0, 496, 1000)

_TC_PIECE = 12500
_TC_NPIECE = _SPLIT // _TC_PIECE
_TC_DEPTH = 6


def _sc_body(ent_in, rel_in, ent_out, rel_out, buf0, buf1, sem_in, sem_out):
    wid = lax.axis_index("s") * _NC + lax.axis_index("c")
    base = _SPLIT + wid * _SC_CHUNK
    bufs = (buf0, buf1)

    def ent_slice(j):
        return pl.ds(base + j * _SC_PIECE, _SC_PIECE)

    in_h = [None, None]
    out_h = [None, None]
    in_h[0] = pltpu.async_copy(
        ent_in.at[ent_slice(0)], bufs[0].at[pl.ds(0, _SC_PIECE)], sem_in)
    for j in range(_SC_NPIECE):
        b = j % 2
        in_h[b].wait()
        if j >= 1:
            out_h[1 - b].wait()
        if j + 1 < _SC_NPIECE:
            in_h[1 - b] = pltpu.async_copy(
                ent_in.at[ent_slice(j + 1)],
                bufs[1 - b].at[pl.ds(0, _SC_PIECE)], sem_in)
        out_h[b] = pltpu.async_copy(
            bufs[b].at[pl.ds(0, _SC_PIECE)],
            ent_out.at[ent_slice(j)], sem_out)
    out_h[(_SC_NPIECE - 1) % 2].wait()

    @pl.when(wid == _NW - 1)
    def _copy_tail():
        sl = pl.ds(_SPLIT + _NW * _SC_CHUNK, _SC_TAIL)
        pltpu.async_copy(ent_in.at[sl], buf0.at[pl.ds(0, _SC_TAIL)], sem_in).wait()
        pltpu.async_copy(buf0.at[pl.ds(0, _SC_TAIL)], ent_out.at[sl], sem_out).wait()

    @pl.when(wid == 0)
    def _copy_rel():
        for k in range(2):
            lo, hi = _REL_SPLIT[k], _REL_SPLIT[k + 1]
            sl = pl.ds(lo, hi - lo)
            pltpu.async_copy(rel_in.at[sl], buf0.at[pl.ds(0, hi - lo)], sem_in).wait()
            pltpu.async_copy(buf0.at[pl.ds(0, hi - lo)], rel_out.at[sl], sem_out).wait()


def _sc_copy(entity_emb, rel_emb):
    fn = pl.kernel(
        _sc_body,
        out_type=(
            jax.ShapeDtypeStruct(entity_emb.shape, entity_emb.dtype),
            jax.ShapeDtypeStruct(rel_emb.shape, rel_emb.dtype),
        ),
        mesh=plsc.VectorSubcoreMesh(core_axis_name="c", subcore_axis_name="s"),
        scratch_types=[
            pltpu.VMEM((_SC_PIECE, 64), jnp.float32),
            pltpu.VMEM((_SC_PIECE, 64), jnp.float32),
            pltpu.SemaphoreType.DMA,
            pltpu.SemaphoreType.DMA,
        ],
    )
    return fn(entity_emb, rel_emb)


def _tc_body(ent_in, half_out, bufs, sem_in, sem_out):
    def sl(j):
        return pl.ds(j * _TC_PIECE, _TC_PIECE)

    in_h = [None] * _TC_DEPTH
    out_h = [None] * _TC_DEPTH
    for b in range(_TC_DEPTH):
        in_h[b] = pltpu.make_async_copy(ent_in.at[sl(b)], bufs.at[b], sem_in)
        in_h[b].start()
    for j in range(_TC_NPIECE):
        b = j % _TC_DEPTH
        in_h[b].wait()
        out_h[b] = pltpu.make_async_copy(bufs.at[b], half_out.at[sl(j)], sem_out)
        out_h[b].start()
        nxt = j + _TC_DEPTH
        if nxt < _TC_NPIECE:
            out_h[b].wait()
            in_h[b] = pltpu.make_async_copy(ent_in.at[sl(nxt)], bufs.at[b], sem_in)
            in_h[b].start()
    for j in range(max(0, _TC_NPIECE - _TC_DEPTH), _TC_NPIECE):
        out_h[j % _TC_DEPTH].wait()


def _tc_copy(entity_emb):
    return pl.pallas_call(
        _tc_body,
        out_shape=jax.ShapeDtypeStruct((_SPLIT, 64), entity_emb.dtype),
        in_specs=[pl.BlockSpec(memory_space=pl.ANY)],
        out_specs=pl.BlockSpec(memory_space=pl.ANY),
        scratch_shapes=[
            pltpu.VMEM((_TC_DEPTH, _TC_PIECE, 64), jnp.float32),
            pltpu.SemaphoreType.DMA,
            pltpu.SemaphoreType.DMA,
        ],
    )(entity_emb)


def kernel(x_dict, edge_index, entity_emb, rel_emb):
    ent_full, rel_out = _sc_copy(entity_emb, rel_emb)
    tc_half = _tc_copy(entity_emb)
    ent_out = lax.dynamic_update_slice(ent_full, tc_half, (0, 0))
    return (ent_out, rel_out)


# speedup vs baseline: 1.0499x; 1.0499x over previous
import jax
import jax.numpy as jnp
from jax import lax
from jax.experimental import pallas as pl
from jax.experimental.pallas import tpu as pltpu

_ENT_ROWS = 1000000
_PIECE = 1000
_NPIECE = _ENT_ROWS // _PIECE
_NBUF = 16
_DEPTH = 8


def _tc_ring_body(ent_in, rel_in, ent_out, rel_out, bufs, relbuf, sem_in, sem_out):
    def start_in(j):
        pltpu.make_async_copy(
            ent_in.at[pl.ds(j * _PIECE, _PIECE)], bufs.at[j % _NBUF], sem_in
        ).start()

    def start_out(j):
        pltpu.make_async_copy(
            bufs.at[j % _NBUF], ent_out.at[pl.ds(j * _PIECE, _PIECE)], sem_out
        ).start()

    def wait_in():
        pltpu.make_async_copy(
            ent_in.at[pl.ds(0, _PIECE)], bufs.at[0], sem_in
        ).wait()

    def wait_out():
        pltpu.make_async_copy(
            bufs.at[0], ent_out.at[pl.ds(0, _PIECE)], sem_out
        ).wait()

    for j in range(_DEPTH):
        start_in(j)

    def step(j, carry):
        wait_in()
        start_out(j)

        @pl.when(j >= _DEPTH)
        def _():
            wait_out()

        @pl.when(j + _DEPTH < _NPIECE)
        def _():
            start_in(j + _DEPTH)

        return carry

    lax.fori_loop(0, _NPIECE, step, 0)
    for _ in range(_DEPTH):
        wait_out()

    rel_in_h = pltpu.make_async_copy(rel_in.at[...], relbuf, sem_in)
    rel_in_h.start()
    rel_in_h.wait()
    rel_out_h = pltpu.make_async_copy(relbuf, rel_out.at[...], sem_out)
    rel_out_h.start()
    rel_out_h.wait()


def kernel(x_dict, edge_index, entity_emb, rel_emb):
    ent_out, rel_out = pl.pallas_call(
        _tc_ring_body,
        out_shape=(
            jax.ShapeDtypeStruct(entity_emb.shape, entity_emb.dtype),
            jax.ShapeDtypeStruct(rel_emb.shape, rel_emb.dtype),
        ),
        in_specs=[
            pl.BlockSpec(memory_space=pl.ANY),
            pl.BlockSpec(memory_space=pl.ANY),
        ],
        out_specs=(
            pl.BlockSpec(memory_space=pl.ANY),
            pl.BlockSpec(memory_space=pl.ANY),
        ),
        scratch_shapes=[
            pltpu.VMEM((_NBUF, _PIECE, 64), jnp.float32),
            pltpu.VMEM((1000, 64), jnp.float32),
            pltpu.SemaphoreType.DMA,
            pltpu.SemaphoreType.DMA,
        ],
    )(entity_emb, rel_emb)
    return (ent_out, rel_out)
